# baseline (speedup 1.0000x reference)
import jax
import jax.numpy as jnp
from jax import lax
from jax.experimental import pallas as pl
from jax.experimental.pallas import tpu as pltpu
from jax.experimental.pallas import tpu_sc as plsc

B = 16384
LREV = 50
D = 64
A = 16
NNZ = 65536
NLAB = 16384
VOCAB = 100000
FL = LREV * D

NC, NS, LANE = 2, 16, 16
NW = NC * NS
PAIR = 2 * LREV
PPW = (B // 2) // NW
PBLK = 32
NPB = PPW // PBLK

_SC_PARAMS = pltpu.CompilerParams(
    needs_layout_passes=False, use_tc_tiling_on_sc=False)


def _sc_mesh():
    return plsc.VectorSubcoreMesh(
        core_axis_name="c", subcore_axis_name="s",
        num_cores=NC, num_subcores=NS)


def _gather_body(hist2, wemb, ewh, idx_blk, bufa, bufb,
                 sga, sgb, swa, swb):
    cid = lax.axis_index("c")
    sid = lax.axis_index("s")
    wid = sid * NC + cid
    pbase = wid * PPW

    def g_start(p, buf, sem):
        pltpu.async_copy(wemb.at[idx_blk.at[p]], buf, sem)

    def g_wait(p, buf, sem):
        pltpu.make_async_copy(wemb.at[idx_blk.at[p]], buf, sem).wait()

    def w_start(pp, buf, sem):
        pltpu.async_copy(buf, ewh.at[pl.ds(pp * PAIR, PAIR)], sem)

    def w_wait(pp, buf, sem):
        pltpu.make_async_copy(
            buf, ewh.at[pl.ds(pp * PAIR, PAIR)], sem).wait()

    @pl.loop(0, NPB)
    def _blk(j):
        pb0 = pbase + j * PBLK
        pltpu.sync_copy(hist2.at[pl.ds(pb0, PBLK)], idx_blk)
        g_start(0, bufa, sga)
        g_start(1, bufb, sgb)

        @pl.loop(0, PBLK, step=2)
        def _p(p):
            g_wait(p, bufa, sga)
            w_start(pb0 + p, bufa, swa)
            g_wait(p + 1, bufb, sgb)
            w_start(pb0 + p + 1, bufb, swb)
            w_wait(pb0 + p, bufa, swa)

            @pl.when(p < PBLK - 2)
            def _():
                g_start(p + 2, bufa, sga)

            w_wait(pb0 + p + 1, bufb, swb)

            @pl.when(p < PBLK - 2)
            def _():
                g_start(p + 3, bufb, sgb)


def _gather_call(hist2, wemb):
    f = pl.kernel(
        _gather_body,
        out_type=jax.ShapeDtypeStruct((B * LREV, D), jnp.bfloat16),
        mesh=_sc_mesh(),
        compiler_params=_SC_PARAMS,
        scratch_types=[
            pltpu.VMEM((PBLK, PAIR), jnp.int32),
            pltpu.VMEM((PAIR, D), jnp.bfloat16),
            pltpu.VMEM((PAIR, D), jnp.bfloat16),
            pltpu.SemaphoreType.DMA,
            pltpu.SemaphoreType.DMA,
            pltpu.SemaphoreType.DMA,
            pltpu.SemaphoreType.DMA,
        ],
    )
    return f(hist2, wemb)


def _v_body(x_ref, m_ref, o_ref):
    o_ref[...] = lax.dot_general(
        x_ref[...], m_ref[...], (((1,), (0,)), ((), ())),
        preferred_element_type=jnp.float32)


def _compute_v(review_positive, M_w):
    blk = B // 8
    return pl.pallas_call(
        _v_body,
        grid=(8,),
        in_specs=[pl.BlockSpec((blk, D), lambda i: (i, 0)),
                  pl.BlockSpec((D, D), lambda i: (0, 0))],
        out_specs=pl.BlockSpec((blk, D), lambda i: (i, 0)),
        out_shape=jax.ShapeDtypeStruct((B, D), jnp.float32),
    )(review_positive, M_w)


RBLK = 128


def _attn_body(ew_ref, v_ref, s64_ref, m3_ref, w_ref, b_ref, o_ref):
    ew = ew_ref[...]
    vrep = jnp.concatenate([v_ref[...].astype(jnp.bfloat16)] * LREV, axis=1)
    dx = lax.dot_general(ew * vrep, s64_ref[...],
                         (((1,), (0,)), ((), ())),
                         preferred_element_type=jnp.float32)
    m = jnp.max(dx, axis=1, keepdims=True)
    es = jnp.exp(dx - m)
    ax = es / jnp.sum(es, axis=1, keepdims=True)
    arep = jnp.concatenate([ax.astype(jnp.bfloat16)] * D, axis=1)
    zs = lax.dot_general(arep * ew, m3_ref[...], (((1,), (0,)), ((), ())),
                         preferred_element_type=jnp.float32)
    o_ref[...] = lax.dot_general(
        zs, w_ref[...], (((1,), (1,)), ((), ())),
        preferred_element_type=jnp.float32) + b_ref[...]


def _attn_call(ew2, v, W_w, W_b2d):
    k = jnp.arange(FL, dtype=jnp.int32)
    s64 = (k[:, None] // D == jnp.arange(LREV)[None, :]).astype(jnp.bfloat16)
    m3 = (k[:, None] // LREV == jnp.arange(D)[None, :]).astype(jnp.bfloat16)
    nblk = B // RBLK
    return pl.pallas_call(
        _attn_body,
        grid=(nblk,),
        in_specs=[pl.BlockSpec((RBLK, FL), lambda i: (i, 0)),
                  pl.BlockSpec((RBLK, D), lambda i: (i, 0)),
                  pl.BlockSpec((FL, LREV), lambda i: (0, 0)),
                  pl.BlockSpec((FL, D), lambda i: (0, 0)),
                  pl.BlockSpec((A, D), lambda i: (0, 0)),
                  pl.BlockSpec((1, A), lambda i: (0, 0))],
        out_specs=pl.BlockSpec((RBLK, A), lambda i: (i, 0)),
        out_shape=jax.ShapeDtypeStruct((B, A), jnp.float32),
    )(ew2, v, s64, m3, W_w, W_b2d)


NZ_PER_TILE = NNZ // NS
CHUNK = 128
NCHUNK = NZ_PER_TILE // CHUNK
ROWS_PER_TILE = NLAB // NS


def _spmm_body(pt, uidx, uval, iidx, ival, uout, iout,
               rows2d, cols2d, vals2d, gat, scl, zrow, acc, sem):
    cid = lax.axis_index("c")
    sid = lax.axis_index("s")
    zero16 = jnp.zeros((LANE,), jnp.float32)

    for i in range(64):
        zrow[i, :] = zero16
    for k in range(ROWS_PER_TILE // 64):
        pltpu.sync_copy(zrow, acc.at[pl.ds(sid * ROWS_PER_TILE + k * 64, 64)])
    plsc.subcore_barrier()

    def process(idx_hbm, val_hbm, out_hbm):
        pltpu.sync_copy(idx_hbm.at[0, pl.ds(sid * NCHUNK, NCHUNK)], rows2d)
        pltpu.sync_copy(idx_hbm.at[1, pl.ds(sid * NCHUNK, NCHUNK)], cols2d)
        pltpu.sync_copy(val_hbm.at[pl.ds(sid * NCHUNK, NCHUNK)], vals2d)

        @pl.loop(0, NCHUNK)
        def _chunk(t):
            pltpu.async_copy(pt.at[cols2d.at[t]], gat, sem).wait()
            vvs = [vals2d[t, pl.ds(16 * k, LANE)] for k in range(CHUNK // 16)]
            for i in range(CHUNK):
                scl[i, :] = gat[i, :] * vvs[i // 16][i % 16]

            pltpu.sync_copy(scl, acc.at[rows2d.at[t]], add=True)

        plsc.subcore_barrier()
        pltpu.sync_copy(acc.at[pl.ds(sid * ROWS_PER_TILE, ROWS_PER_TILE)],
                        out_hbm.at[pl.ds(sid * ROWS_PER_TILE, ROWS_PER_TILE)])

    @pl.when(cid == 0)
    def _():
        process(uidx, uval, uout)

    @pl.when(cid == 1)
    def _():
        process(iidx, ival, iout)


def _spmm_call(pt, uidx, uval, iidx, ival):
    f = pl.kernel(
        _spmm_body,
        out_type=(jax.ShapeDtypeStruct((NLAB, A), jnp.float32),
                  jax.ShapeDtypeStruct((NLAB, A), jnp.float32)),
        mesh=_sc_mesh(),
        compiler_params=_SC_PARAMS,
        scratch_types=[
            pltpu.VMEM((NCHUNK, CHUNK), jnp.int32),
            pltpu.VMEM((NCHUNK, CHUNK), jnp.int32),
            pltpu.VMEM((NCHUNK, CHUNK), jnp.float32),
            pltpu.VMEM((CHUNK, A), jnp.float32),
            pltpu.VMEM((CHUNK, A), jnp.float32),
            pltpu.VMEM((64, A), jnp.float32),
            pltpu.VMEM_SHARED((NLAB, A), jnp.float32),
            pltpu.SemaphoreType.DMA,
        ],
    )
    return f(pt, uidx, uval, iidx, ival)


def kernel(historical_review, review_positive, review_negative,
           user_histor_index, user_histor_value,
           item_histor_index, item_histor_value,
           word_embedding, M_w, W_w, W_b, T_w):
    hist2 = historical_review.astype(jnp.int32).reshape(B // 2, PAIR)
    uidx = user_histor_index.astype(jnp.int32).reshape(2, NNZ // CHUNK, CHUNK)
    iidx = item_histor_index.astype(jnp.int32).reshape(2, NNZ // CHUNK, CHUNK)
    uval = user_histor_value.reshape(NNZ // CHUNK, CHUNK)
    ival = item_histor_value.reshape(NNZ // CHUNK, CHUNK)

    ew = _gather_call(hist2, word_embedding.astype(jnp.bfloat16))
    v = _compute_v(review_positive, M_w)
    pt = _attn_call(ew.reshape(B, FL), v, W_w, W_b.reshape(1, A))
    return _spmm_call(pt, uidx, uval, iidx, ival)


# speedup vs baseline: 1.1748x; 1.1748x over previous
import jax
import jax.numpy as jnp
from jax import lax
from jax.experimental import pallas as pl
from jax.experimental.pallas import tpu as pltpu
from jax.experimental.pallas import tpu_sc as plsc

B = 16384
LREV = 50
D = 64
A = 16
NNZ = 65536
NLAB = 16384
VOCAB = 100000
FL = LREV * D

NC, NS, LANE = 2, 16, 16
NW = NC * NS
PAIR = 2 * LREV
PPW = (B // 2) // NW
PBLK = 32
NPB = PPW // PBLK

_SC_PARAMS = pltpu.CompilerParams(
    needs_layout_passes=False, use_tc_tiling_on_sc=False)


def _sc_mesh():
    return plsc.VectorSubcoreMesh(
        core_axis_name="c", subcore_axis_name="s",
        num_cores=NC, num_subcores=NS)


def _gather_body(hist2, wemb, ewh, idx_blk, bufa, bufb,
                 sga, sgb, swa, swb):
    cid = lax.axis_index("c")
    sid = lax.axis_index("s")
    wid = sid * NC + cid
    pbase = wid * PPW

    def g_start(p, buf, sem):
        pltpu.async_copy(wemb.at[idx_blk.at[p]], buf, sem)

    def g_wait(p, buf, sem):
        pltpu.make_async_copy(wemb.at[idx_blk.at[p]], buf, sem).wait()

    def w_start(pp, buf, sem):
        pltpu.async_copy(buf, ewh.at[pl.ds(pp * PAIR, PAIR)], sem)

    def w_wait(pp, buf, sem):
        pltpu.make_async_copy(
            buf, ewh.at[pl.ds(pp * PAIR, PAIR)], sem).wait()

    @pl.loop(0, NPB)
    def _blk(j):
        pb0 = pbase + j * PBLK
        pltpu.sync_copy(hist2.at[pl.ds(pb0, PBLK)], idx_blk)
        g_start(0, bufa, sga)
        g_start(1, bufb, sgb)

        @pl.loop(0, PBLK, step=2)
        def _p(p):
            g_wait(p, bufa, sga)
            w_start(pb0 + p, bufa, swa)
            g_wait(p + 1, bufb, sgb)
            w_start(pb0 + p + 1, bufb, swb)
            w_wait(pb0 + p, bufa, swa)

            @pl.when(p < PBLK - 2)
            def _():
                g_start(p + 2, bufa, sga)

            w_wait(pb0 + p + 1, bufb, swb)

            @pl.when(p < PBLK - 2)
            def _():
                g_start(p + 3, bufb, sgb)


def _gather_call(hist2, wemb):
    f = pl.kernel(
        _gather_body,
        out_type=jax.ShapeDtypeStruct((B * LREV, D), jnp.float32),
        mesh=_sc_mesh(),
        compiler_params=_SC_PARAMS,
        scratch_types=[
            pltpu.VMEM((PBLK, PAIR), jnp.int32),
            pltpu.VMEM((PAIR, D), jnp.float32),
            pltpu.VMEM((PAIR, D), jnp.float32),
            pltpu.SemaphoreType.DMA,
            pltpu.SemaphoreType.DMA,
            pltpu.SemaphoreType.DMA,
            pltpu.SemaphoreType.DMA,
        ],
    )
    return f(hist2, wemb)


def _v_body(x_ref, m_ref, o_ref):
    o_ref[...] = lax.dot_general(
        x_ref[...], m_ref[...], (((1,), (0,)), ((), ())),
        preferred_element_type=jnp.float32)


def _compute_v(review_positive, M_w):
    blk = B // 8
    return pl.pallas_call(
        _v_body,
        grid=(8,),
        in_specs=[pl.BlockSpec((blk, D), lambda i: (i, 0)),
                  pl.BlockSpec((D, D), lambda i: (0, 0))],
        out_specs=pl.BlockSpec((blk, D), lambda i: (i, 0)),
        out_shape=jax.ShapeDtypeStruct((B, D), jnp.float32),
    )(review_positive, M_w)


RBLK = 128


def _attn_body(ew_ref, v_ref, s64_ref, m3_ref, w_ref, b_ref, o_ref):
    ew = ew_ref[...]
    vrep = jnp.concatenate([v_ref[...]] * LREV, axis=1)
    dx = lax.dot_general(ew * vrep, s64_ref[...],
                         (((1,), (0,)), ((), ())),
                         preferred_element_type=jnp.float32)
    m = jnp.max(dx, axis=1, keepdims=True)
    es = jnp.exp(dx - m)
    ax = es / jnp.sum(es, axis=1, keepdims=True)
    arep = jnp.concatenate([ax] * D, axis=1)
    zs = lax.dot_general(arep * ew, m3_ref[...], (((1,), (0,)), ((), ())),
                         preferred_element_type=jnp.float32)
    o_ref[...] = lax.dot_general(
        zs, w_ref[...], (((1,), (1,)), ((), ())),
        preferred_element_type=jnp.float32) + b_ref[...]


def _attn_call(ew2, v, W_w, W_b2d):
    k = jnp.arange(FL, dtype=jnp.int32)
    s64 = (k[:, None] // D == jnp.arange(LREV)[None, :]).astype(jnp.float32)
    m3 = (k[:, None] // LREV == jnp.arange(D)[None, :]).astype(jnp.float32)
    nblk = B // RBLK
    return pl.pallas_call(
        _attn_body,
        grid=(nblk,),
        in_specs=[pl.BlockSpec((RBLK, FL), lambda i: (i, 0)),
                  pl.BlockSpec((RBLK, D), lambda i: (i, 0)),
                  pl.BlockSpec((FL, LREV), lambda i: (0, 0)),
                  pl.BlockSpec((FL, D), lambda i: (0, 0)),
                  pl.BlockSpec((A, D), lambda i: (0, 0)),
                  pl.BlockSpec((1, A), lambda i: (0, 0))],
        out_specs=pl.BlockSpec((RBLK, A), lambda i: (i, 0)),
        out_shape=jax.ShapeDtypeStruct((B, A), jnp.float32),
    )(ew2, v, s64, m3, W_w, W_b2d)


NZ_PER_TILE = NNZ // NS
CHUNK = 128
NCHUNK = NZ_PER_TILE // CHUNK
ROWS_PER_TILE = NLAB // NS


def _spmm_body(pt, uidx, uval, iidx, ival, uout, iout,
               rows2d, cols2d, vals2d, gat, scl, zrow, acc, sem):
    cid = lax.axis_index("c")
    sid = lax.axis_index("s")
    zero16 = jnp.zeros((LANE,), jnp.float32)

    for i in range(64):
        zrow[i, :] = zero16
    for k in range(ROWS_PER_TILE // 64):
        pltpu.sync_copy(zrow, acc.at[pl.ds(sid * ROWS_PER_TILE + k * 64, 64)])
    plsc.subcore_barrier()

    def process(idx_hbm, val_hbm, out_hbm):
        pltpu.sync_copy(idx_hbm.at[0, pl.ds(sid * NCHUNK, NCHUNK)], rows2d)
        pltpu.sync_copy(idx_hbm.at[1, pl.ds(sid * NCHUNK, NCHUNK)], cols2d)
        pltpu.sync_copy(val_hbm.at[pl.ds(sid * NCHUNK, NCHUNK)], vals2d)

        @pl.loop(0, NCHUNK)
        def _chunk(t):
            pltpu.async_copy(pt.at[cols2d.at[t]], gat, sem).wait()
            vvs = [vals2d[t, pl.ds(16 * k, LANE)] for k in range(CHUNK // 16)]
            for i in range(CHUNK):
                scl[i, :] = gat[i, :] * vvs[i // 16][i % 16]

            pltpu.sync_copy(scl, acc.at[rows2d.at[t]], add=True)

        plsc.subcore_barrier()
        pltpu.sync_copy(acc.at[pl.ds(sid * ROWS_PER_TILE, ROWS_PER_TILE)],
                        out_hbm.at[pl.ds(sid * ROWS_PER_TILE, ROWS_PER_TILE)])

    @pl.when(cid == 0)
    def _():
        process(uidx, uval, uout)

    @pl.when(cid == 1)
    def _():
        process(iidx, ival, iout)


def _spmm_call(pt, uidx, uval, iidx, ival):
    f = pl.kernel(
        _spmm_body,
        out_type=(jax.ShapeDtypeStruct((NLAB, A), jnp.float32),
                  jax.ShapeDtypeStruct((NLAB, A), jnp.float32)),
        mesh=_sc_mesh(),
        compiler_params=_SC_PARAMS,
        scratch_types=[
            pltpu.VMEM((NCHUNK, CHUNK), jnp.int32),
            pltpu.VMEM((NCHUNK, CHUNK), jnp.int32),
            pltpu.VMEM((NCHUNK, CHUNK), jnp.float32),
            pltpu.VMEM((CHUNK, A), jnp.float32),
            pltpu.VMEM((CHUNK, A), jnp.float32),
            pltpu.VMEM((64, A), jnp.float32),
            pltpu.VMEM_SHARED((NLAB, A), jnp.float32),
            pltpu.SemaphoreType.DMA,
        ],
    )
    return f(pt, uidx, uval, iidx, ival)


def kernel(historical_review, review_positive, review_negative,
           user_histor_index, user_histor_value,
           item_histor_index, item_histor_value,
           word_embedding, M_w, W_w, W_b, T_w):
    hist2 = historical_review.astype(jnp.int32).reshape(B // 2, PAIR)
    uidx = user_histor_index.astype(jnp.int32).reshape(2, NNZ // CHUNK, CHUNK)
    iidx = item_histor_index.astype(jnp.int32).reshape(2, NNZ // CHUNK, CHUNK)
    uval = user_histor_value.reshape(NNZ // CHUNK, CHUNK)
    ival = item_histor_value.reshape(NNZ // CHUNK, CHUNK)

    ew = _gather_call(hist2, word_embedding)
    v = _compute_v(review_positive, M_w)
    pt = _attn_call(ew.reshape(B, FL), v, W_w, W_b.reshape(1, A))
    return _spmm_call(pt, uidx, uval, iidx, ival)


# speedup vs baseline: 1.2779x; 1.0878x over previous
import jax
import jax.numpy as jnp
from jax import lax
from jax.experimental import pallas as pl
from jax.experimental.pallas import tpu as pltpu
from jax.experimental.pallas import tpu_sc as plsc

B = 16384
LREV = 50
D = 64
A = 16
NNZ = 65536
NLAB = 16384
VOCAB = 100000
FL = LREV * D

NC, NS, LANE = 2, 16, 16
NW = NC * NS
PAIR = 2 * LREV
PPW = (B // 2) // NW
PBLK = 32
NPB = PPW // PBLK

_SC_PARAMS = pltpu.CompilerParams(
    needs_layout_passes=False, use_tc_tiling_on_sc=False)


def _sc_mesh():
    return plsc.VectorSubcoreMesh(
        core_axis_name="c", subcore_axis_name="s",
        num_cores=NC, num_subcores=NS)


def _gather_body(hist2, wemb, ewh, idx_blk, bufa, bufb,
                 sga, sgb, swa, swb):
    cid = lax.axis_index("c")
    sid = lax.axis_index("s")
    wid = sid * NC + cid
    pbase = wid * PPW

    def g_start(p, buf, sem):
        pltpu.async_copy(wemb.at[idx_blk.at[p]], buf, sem)

    def g_wait(p, buf, sem):
        pltpu.make_async_copy(wemb.at[idx_blk.at[p]], buf, sem).wait()

    def w_start(pp, buf, sem):
        pltpu.async_copy(buf, ewh.at[pl.ds(pp * PAIR, PAIR)], sem)

    def w_wait(pp, buf, sem):
        pltpu.make_async_copy(
            buf, ewh.at[pl.ds(pp * PAIR, PAIR)], sem).wait()

    @pl.loop(0, NPB)
    def _blk(j):
        pb0 = pbase + j * PBLK
        pltpu.sync_copy(hist2.at[pl.ds(pb0, PBLK)], idx_blk)
        g_start(0, bufa, sga)
        g_start(1, bufb, sgb)

        @pl.loop(0, PBLK, step=2)
        def _p(p):
            g_wait(p, bufa, sga)
            w_start(pb0 + p, bufa, swa)
            g_wait(p + 1, bufb, sgb)
            w_start(pb0 + p + 1, bufb, swb)
            w_wait(pb0 + p, bufa, swa)

            @pl.when(p < PBLK - 2)
            def _():
                g_start(p + 2, bufa, sga)

            w_wait(pb0 + p + 1, bufb, swb)

            @pl.when(p < PBLK - 2)
            def _():
                g_start(p + 3, bufb, sgb)


def _gather_call(hist2, wemb):
    f = pl.kernel(
        _gather_body,
        out_type=jax.ShapeDtypeStruct((B * LREV, D), jnp.float32),
        mesh=_sc_mesh(),
        compiler_params=_SC_PARAMS,
        scratch_types=[
            pltpu.VMEM((PBLK, PAIR), jnp.int32),
            pltpu.VMEM((PAIR, D), jnp.float32),
            pltpu.VMEM((PAIR, D), jnp.float32),
            pltpu.SemaphoreType.DMA,
            pltpu.SemaphoreType.DMA,
            pltpu.SemaphoreType.DMA,
            pltpu.SemaphoreType.DMA,
        ],
    )
    return f(hist2, wemb)


def _v_body(x_ref, m_ref, o_ref):
    o_ref[...] = lax.dot_general(
        x_ref[...], m_ref[...], (((1,), (0,)), ((), ())),
        preferred_element_type=jnp.float32)


def _compute_v(review_positive, M_w):
    blk = B // 8
    return pl.pallas_call(
        _v_body,
        grid=(8,),
        in_specs=[pl.BlockSpec((blk, D), lambda i: (i, 0)),
                  pl.BlockSpec((D, D), lambda i: (0, 0))],
        out_specs=pl.BlockSpec((blk, D), lambda i: (i, 0)),
        out_shape=jax.ShapeDtypeStruct((B, D), jnp.float32),
    )(review_positive, M_w)


RBLK = 128


def _attn_body(ew_ref, v_ref, ve_ref, s64_ref, e2_ref, m3_ref,
               w_ref, b_ref, o_ref):
    ew = ew_ref[...]
    vrep = lax.dot_general(v_ref[...], ve_ref[...],
                           (((1,), (0,)), ((), ())),
                           preferred_element_type=jnp.float32)
    dx = lax.dot_general(ew * vrep, s64_ref[...],
                         (((1,), (0,)), ((), ())),
                         preferred_element_type=jnp.float32)
    m = jnp.max(dx, axis=1, keepdims=True)
    es = jnp.exp(dx - m)
    ax = es / jnp.sum(es, axis=1, keepdims=True)
    arep = lax.dot_general(ax, e2_ref[...], (((1,), (0,)), ((), ())),
                           preferred_element_type=jnp.float32)
    zs = lax.dot_general(arep * ew, m3_ref[...], (((1,), (0,)), ((), ())),
                         preferred_element_type=jnp.float32)
    o_ref[...] = lax.dot_general(
        zs, w_ref[...], (((1,), (1,)), ((), ())),
        preferred_element_type=jnp.float32) + b_ref[...]


def _attn_call(ew2, v, W_w, W_b2d):
    k = jnp.arange(FL, dtype=jnp.int32)
    ve = (k[None, :] % D == jnp.arange(D)[:, None]).astype(jnp.float32)
    s64 = (k[:, None] // D == jnp.arange(LREV)[None, :]).astype(jnp.float32)
    e2 = (k[None, :] % LREV == jnp.arange(LREV)[:, None]).astype(jnp.float32)
    m3 = (k[:, None] // LREV == jnp.arange(D)[None, :]).astype(jnp.float32)
    nblk = B // RBLK
    return pl.pallas_call(
        _attn_body,
        grid=(nblk,),
        in_specs=[pl.BlockSpec((RBLK, FL), lambda i: (i, 0)),
                  pl.BlockSpec((RBLK, D), lambda i: (i, 0)),
                  pl.BlockSpec((D, FL), lambda i: (0, 0)),
                  pl.BlockSpec((FL, LREV), lambda i: (0, 0)),
                  pl.BlockSpec((LREV, FL), lambda i: (0, 0)),
                  pl.BlockSpec((FL, D), lambda i: (0, 0)),
                  pl.BlockSpec((A, D), lambda i: (0, 0)),
                  pl.BlockSpec((1, A), lambda i: (0, 0))],
        out_specs=pl.BlockSpec((RBLK, A), lambda i: (i, 0)),
        out_shape=jax.ShapeDtypeStruct((B, A), jnp.float32),
    )(ew2, v, ve, s64, e2, m3, W_w, W_b2d)


NZ_PER_TILE = NNZ // NS
CHUNK = 128
NCHUNK = NZ_PER_TILE // CHUNK
ROWS_PER_TILE = NLAB // NS


def _spmm_body(pt, uidx, uval, iidx, ival, uout, iout,
               rows2d, cols2d, vals2d, gat, scl, zrow, acc, sem):
    cid = lax.axis_index("c")
    sid = lax.axis_index("s")
    zero16 = jnp.zeros((LANE,), jnp.float32)

    for i in range(64):
        zrow[i, :] = zero16
    for k in range(ROWS_PER_TILE // 64):
        pltpu.sync_copy(zrow, acc.at[pl.ds(sid * ROWS_PER_TILE + k * 64, 64)])
    plsc.subcore_barrier()

    def process(idx_hbm, val_hbm, out_hbm):
        pltpu.sync_copy(idx_hbm.at[0, pl.ds(sid * NCHUNK, NCHUNK)], rows2d)
        pltpu.sync_copy(idx_hbm.at[1, pl.ds(sid * NCHUNK, NCHUNK)], cols2d)
        pltpu.sync_copy(val_hbm.at[pl.ds(sid * NCHUNK, NCHUNK)], vals2d)

        @pl.loop(0, NCHUNK)
        def _chunk(t):
            pltpu.async_copy(pt.at[cols2d.at[t]], gat, sem).wait()
            vvs = [vals2d[t, pl.ds(16 * k, LANE)] for k in range(CHUNK // 16)]
            for i in range(CHUNK):
                scl[i, :] = gat[i, :] * vvs[i // 16][i % 16]

            pltpu.sync_copy(scl, acc.at[rows2d.at[t]], add=True)

        plsc.subcore_barrier()
        pltpu.sync_copy(acc.at[pl.ds(sid * ROWS_PER_TILE, ROWS_PER_TILE)],
                        out_hbm.at[pl.ds(sid * ROWS_PER_TILE, ROWS_PER_TILE)])

    @pl.when(cid == 0)
    def _():
        process(uidx, uval, uout)

    @pl.when(cid == 1)
    def _():
        process(iidx, ival, iout)


def _spmm_call(pt, uidx, uval, iidx, ival):
    f = pl.kernel(
        _spmm_body,
        out_type=(jax.ShapeDtypeStruct((NLAB, A), jnp.float32),
                  jax.ShapeDtypeStruct((NLAB, A), jnp.float32)),
        mesh=_sc_mesh(),
        compiler_params=_SC_PARAMS,
        scratch_types=[
            pltpu.VMEM((NCHUNK, CHUNK), jnp.int32),
            pltpu.VMEM((NCHUNK, CHUNK), jnp.int32),
            pltpu.VMEM((NCHUNK, CHUNK), jnp.float32),
            pltpu.VMEM((CHUNK, A), jnp.float32),
            pltpu.VMEM((CHUNK, A), jnp.float32),
            pltpu.VMEM((64, A), jnp.float32),
            pltpu.VMEM_SHARED((NLAB, A), jnp.float32),
            pltpu.SemaphoreType.DMA,
        ],
    )
    return f(pt, uidx, uval, iidx, ival)


def kernel(historical_review, review_positive, review_negative,
           user_histor_index, user_histor_value,
           item_histor_index, item_histor_value,
           word_embedding, M_w, W_w, W_b, T_w):
    hist2 = historical_review.astype(jnp.int32).reshape(B // 2, PAIR)
    uidx = user_histor_index.astype(jnp.int32).reshape(2, NNZ // CHUNK, CHUNK)
    iidx = item_histor_index.astype(jnp.int32).reshape(2, NNZ // CHUNK, CHUNK)
    uval = user_histor_value.reshape(NNZ // CHUNK, CHUNK)
    ival = item_histor_value.reshape(NNZ // CHUNK, CHUNK)

    ew = _gather_call(hist2, word_embedding)
    v = _compute_v(review_positive, M_w)
    pt = _attn_call(ew.reshape(B, FL), v, W_w, W_b.reshape(1, A))
    return _spmm_call(pt, uidx, uval, iidx, ival)


# speedup vs baseline: 1.3434x; 1.0512x over previous
import jax
import jax.numpy as jnp
from jax import lax
from jax.experimental import pallas as pl
from jax.experimental.pallas import tpu as pltpu
from jax.experimental.pallas import tpu_sc as plsc

B = 16384
LREV = 50
D = 64
A = 16
NNZ = 65536
NLAB = 16384
VOCAB = 100000
FL = LREV * D

NC, NS, LANE = 2, 16, 16
NW = NC * NS
PAIR = 2 * LREV
PPW = (B // 2) // NW
PBLK = 32
NPB = PPW // PBLK

_SC_PARAMS = pltpu.CompilerParams(
    needs_layout_passes=False, use_tc_tiling_on_sc=False)


def _sc_mesh():
    return plsc.VectorSubcoreMesh(
        core_axis_name="c", subcore_axis_name="s",
        num_cores=NC, num_subcores=NS)


def _gather_body(hist2, wemb, ewh, idx_blk, bufa, bufb,
                 sga, sgb, swa, swb):
    cid = lax.axis_index("c")
    sid = lax.axis_index("s")
    wid = sid * NC + cid
    pbase = wid * PPW

    def g_start(p, buf, sem):
        pltpu.async_copy(wemb.at[idx_blk.at[p]], buf, sem)

    def g_wait(p, buf, sem):
        pltpu.make_async_copy(wemb.at[idx_blk.at[p]], buf, sem).wait()

    def w_start(pp, buf, sem):
        pltpu.async_copy(buf, ewh.at[pl.ds(pp * PAIR, PAIR)], sem)

    def w_wait(pp, buf, sem):
        pltpu.make_async_copy(
            buf, ewh.at[pl.ds(pp * PAIR, PAIR)], sem).wait()

    @pl.loop(0, NPB)
    def _blk(j):
        pb0 = pbase + j * PBLK
        pltpu.sync_copy(hist2.at[pl.ds(pb0, PBLK)], idx_blk)
        g_start(0, bufa, sga)
        g_start(1, bufb, sgb)

        @pl.loop(0, PBLK, step=2)
        def _p(p):
            g_wait(p, bufa, sga)
            w_start(pb0 + p, bufa, swa)
            g_wait(p + 1, bufb, sgb)
            w_start(pb0 + p + 1, bufb, swb)
            w_wait(pb0 + p, bufa, swa)

            @pl.when(p < PBLK - 2)
            def _():
                g_start(p + 2, bufa, sga)

            w_wait(pb0 + p + 1, bufb, swb)

            @pl.when(p < PBLK - 2)
            def _():
                g_start(p + 3, bufb, sgb)


def _gather_call(hist2, wemb):
    f = pl.kernel(
        _gather_body,
        out_type=jax.ShapeDtypeStruct((B * LREV, D), jnp.float32),
        mesh=_sc_mesh(),
        compiler_params=_SC_PARAMS,
        scratch_types=[
            pltpu.VMEM((PBLK, PAIR), jnp.int32),
            pltpu.VMEM((PAIR, D), jnp.float32),
            pltpu.VMEM((PAIR, D), jnp.float32),
            pltpu.SemaphoreType.DMA,
            pltpu.SemaphoreType.DMA,
            pltpu.SemaphoreType.DMA,
            pltpu.SemaphoreType.DMA,
        ],
    )
    return f(hist2, wemb)


def _v_body(x_ref, m_ref, o_ref):
    o_ref[...] = lax.dot_general(
        x_ref[...], m_ref[...], (((1,), (0,)), ((), ())),
        preferred_element_type=jnp.float32)


def _compute_v(review_positive, M_w):
    blk = B // 8
    return pl.pallas_call(
        _v_body,
        grid=(8,),
        in_specs=[pl.BlockSpec((blk, D), lambda i: (i, 0)),
                  pl.BlockSpec((D, D), lambda i: (0, 0))],
        out_specs=pl.BlockSpec((blk, D), lambda i: (i, 0)),
        out_shape=jax.ShapeDtypeStruct((B, D), jnp.float32),
    )(review_positive, M_w)


RBLK = 256


def _attn_body(ew_ref, v_ref, ve_ref, s64_ref, e2_ref, m3_ref,
               w_ref, b_ref, o_ref):
    ew = ew_ref[...]
    vrep = lax.dot_general(v_ref[...], ve_ref[...],
                           (((1,), (0,)), ((), ())),
                           preferred_element_type=jnp.float32)
    dx = lax.dot_general(ew * vrep, s64_ref[...],
                         (((1,), (0,)), ((), ())),
                         preferred_element_type=jnp.float32)
    m = jnp.max(dx, axis=1, keepdims=True)
    es = jnp.exp(dx - m)
    ax = es / jnp.sum(es, axis=1, keepdims=True)
    arep = lax.dot_general(ax, e2_ref[...], (((1,), (0,)), ((), ())),
                           preferred_element_type=jnp.float32)
    zs = lax.dot_general(arep * ew, m3_ref[...], (((1,), (0,)), ((), ())),
                         preferred_element_type=jnp.float32)
    o_ref[...] = lax.dot_general(
        zs, w_ref[...], (((1,), (1,)), ((), ())),
        preferred_element_type=jnp.float32) + b_ref[...]


def _attn_call(ew2, v, W_w, W_b2d):
    k = jnp.arange(FL, dtype=jnp.int32)
    ve = (k[None, :] % D == jnp.arange(D)[:, None]).astype(jnp.float32)
    s64 = (k[:, None] // D == jnp.arange(LREV)[None, :]).astype(jnp.float32)
    e2 = (k[None, :] % LREV == jnp.arange(LREV)[:, None]).astype(jnp.float32)
    m3 = (k[:, None] // LREV == jnp.arange(D)[None, :]).astype(jnp.float32)
    nblk = B // RBLK
    return pl.pallas_call(
        _attn_body,
        grid=(nblk,),
        in_specs=[pl.BlockSpec((RBLK, FL), lambda i: (i, 0)),
                  pl.BlockSpec((RBLK, D), lambda i: (i, 0)),
                  pl.BlockSpec((D, FL), lambda i: (0, 0)),
                  pl.BlockSpec((FL, LREV), lambda i: (0, 0)),
                  pl.BlockSpec((LREV, FL), lambda i: (0, 0)),
                  pl.BlockSpec((FL, D), lambda i: (0, 0)),
                  pl.BlockSpec((A, D), lambda i: (0, 0)),
                  pl.BlockSpec((1, A), lambda i: (0, 0))],
        out_specs=pl.BlockSpec((RBLK, A), lambda i: (i, 0)),
        out_shape=jax.ShapeDtypeStruct((B, A), jnp.float32),
    )(ew2, v, ve, s64, e2, m3, W_w, W_b2d)


NZ_PER_TILE = NNZ // NS
CHUNK = 128
NCHUNK = NZ_PER_TILE // CHUNK
ROWS_PER_TILE = NLAB // NS


def _spmm_body(pt, uidx, uval, iidx, ival, uout, iout,
               rows2d, cols2d, vals2d, gat, scl, zrow, acc, sem):
    cid = lax.axis_index("c")
    sid = lax.axis_index("s")
    zero16 = jnp.zeros((LANE,), jnp.float32)

    for i in range(64):
        zrow[i, :] = zero16
    for k in range(ROWS_PER_TILE // 64):
        pltpu.sync_copy(zrow, acc.at[pl.ds(sid * ROWS_PER_TILE + k * 64, 64)])
    plsc.subcore_barrier()

    def process(idx_hbm, val_hbm, out_hbm):
        pltpu.sync_copy(idx_hbm.at[0, pl.ds(sid * NCHUNK, NCHUNK)], rows2d)
        pltpu.sync_copy(idx_hbm.at[1, pl.ds(sid * NCHUNK, NCHUNK)], cols2d)
        pltpu.sync_copy(val_hbm.at[pl.ds(sid * NCHUNK, NCHUNK)], vals2d)

        @pl.loop(0, NCHUNK)
        def _chunk(t):
            pltpu.async_copy(pt.at[cols2d.at[t]], gat, sem).wait()
            vvs = [vals2d[t, pl.ds(16 * k, LANE)] for k in range(CHUNK // 16)]
            for i in range(CHUNK):
                scl[i, :] = gat[i, :] * vvs[i // 16][i % 16]

            pltpu.sync_copy(scl, acc.at[rows2d.at[t]], add=True)

        plsc.subcore_barrier()
        pltpu.sync_copy(acc.at[pl.ds(sid * ROWS_PER_TILE, ROWS_PER_TILE)],
                        out_hbm.at[pl.ds(sid * ROWS_PER_TILE, ROWS_PER_TILE)])

    @pl.when(cid == 0)
    def _():
        process(uidx, uval, uout)

    @pl.when(cid == 1)
    def _():
        process(iidx, ival, iout)


def _spmm_call(pt, uidx, uval, iidx, ival):
    f = pl.kernel(
        _spmm_body,
        out_type=(jax.ShapeDtypeStruct((NLAB, A), jnp.float32),
                  jax.ShapeDtypeStruct((NLAB, A), jnp.float32)),
        mesh=_sc_mesh(),
        compiler_params=_SC_PARAMS,
        scratch_types=[
            pltpu.VMEM((NCHUNK, CHUNK), jnp.int32),
            pltpu.VMEM((NCHUNK, CHUNK), jnp.int32),
            pltpu.VMEM((NCHUNK, CHUNK), jnp.float32),
            pltpu.VMEM((CHUNK, A), jnp.float32),
            pltpu.VMEM((CHUNK, A), jnp.float32),
            pltpu.VMEM((64, A), jnp.float32),
            pltpu.VMEM_SHARED((NLAB, A), jnp.float32),
            pltpu.SemaphoreType.DMA,
        ],
    )
    return f(pt, uidx, uval, iidx, ival)


def kernel(historical_review, review_positive, review_negative,
           user_histor_index, user_histor_value,
           item_histor_index, item_histor_value,
           word_embedding, M_w, W_w, W_b, T_w):
    hist2 = historical_review.astype(jnp.int32).reshape(B // 2, PAIR)
    uidx = user_histor_index.astype(jnp.int32).reshape(2, NNZ // CHUNK, CHUNK)
    iidx = item_histor_index.astype(jnp.int32).reshape(2, NNZ // CHUNK, CHUNK)
    uval = user_histor_value.reshape(NNZ // CHUNK, CHUNK)
    ival = item_histor_value.reshape(NNZ // CHUNK, CHUNK)

    ew = _gather_call(hist2, word_embedding)
    v = _compute_v(review_positive, M_w)
    pt = _attn_call(ew.reshape(B, FL), v, W_w, W_b.reshape(1, A))
    return _spmm_call(pt, uidx, uval, iidx, ival)
